# baseline (speedup 1.0000x reference)
import functools

import jax
import jax.numpy as jnp
from jax import lax
from jax.experimental import pallas as pl
from jax.experimental.pallas import tpu as pltpu
from jax.experimental.pallas import tpu_sc as plsc

_N = 20000
_PAD = 20480
_R = 160
_C = 128
_PRE = 2000
_POST = 1000
_TH = 0.7
_NEG = -1e30
_IMG_H = 1024.0
_IMG_W = 1024.0
_INT_MIN = -2147483648
_INT_MAX = 2147483647

_CH = _PAD // 16
_SLOTS = 2048
_TRASH = _SLOTS
_GROWS = _SLOTS + 16


def _prep_body(a0, a1, a2, a3, d0, d1, d2, d3, s,
               x1o, y1o, x2o, y2o, mso, poolo):
    A0, A1, A2, A3 = a0[...], a1[...], a2[...], a3[...]
    D0, D1, D2, D3 = d0[...], d1[...], d2[...], d3[...]
    S = s[...]

    ctr_x = (A0 + A2) / 2.0
    ctr_y = (A1 + A3) / 2.0
    w = A2 - A0
    h = A3 - A1
    x1 = jnp.minimum(jnp.maximum(ctr_x - D0 * w, 0.0), _IMG_W)
    y1 = jnp.minimum(jnp.maximum(ctr_y - D1 * h, 0.0), _IMG_H)
    x2 = jnp.minimum(jnp.maximum(ctr_x + D2 * w, 0.0), _IMG_W)
    y2 = jnp.minimum(jnp.maximum(ctr_y + D3 * h, 0.0), _IMG_H)

    row = lax.broadcasted_iota(jnp.int32, (_R, _C), 0)
    col = lax.broadcasted_iota(jnp.int32, (_R, _C), 1)
    idx = row * _C + col
    inb = idx < _N

    valid = (x2 - x1 > 0.0) & (y2 - y1 > 0.0) & inb
    ms = jnp.where(valid, S, _NEG)

    b = lax.bitcast_convert_type(ms, jnp.int32)
    key = jnp.where(b >= 0, b, _INT_MIN - b)

    def bs1(_, lohi):
        lo, hi = lohi
        mid = (lo >> 1) + (hi >> 1) + (lo & hi & 1)
        cnt = jnp.sum((inb & (key > mid)).astype(jnp.int32))
        go_hi = cnt < _PRE
        live = lo < hi
        new_lo = jnp.where(live & ~go_hi, mid + 1, lo)
        new_hi = jnp.where(live & go_hi, mid, hi)
        return new_lo, new_hi

    tau, _unused_hi = lax.fori_loop(
        0, 32, bs1, (jnp.int32(_INT_MIN), jnp.int32(_INT_MAX)))

    n_gt = jnp.sum((inb & (key > tau)).astype(jnp.int32))
    quota = _PRE - n_gt
    tie = inb & (key == tau)

    def bs2(_, lohi):
        lo, hi = lohi
        mid = (lo + hi) // 2
        cnt = jnp.sum((tie & (idx < mid)).astype(jnp.int32))
        go_hi = cnt >= quota
        live = lo < hi
        new_lo = jnp.where(live & ~go_hi, mid + 1, lo)
        new_hi = jnp.where(live & go_hi, mid, hi)
        return new_lo, new_hi

    mcut, _unused_hi2 = lax.fori_loop(0, 15, bs2, (jnp.int32(0), jnp.int32(_PAD)))

    pool = inb & ((key > tau) | (tie & (idx < mcut)))

    x1o[...] = x1
    y1o[...] = y1
    x2o[...] = x2
    y2o[...] = y2
    mso[...] = ms
    poolo[...] = pool.astype(jnp.int32)


_prep_call = pl.pallas_call(
    _prep_body,
    out_shape=[jax.ShapeDtypeStruct((_R, _C), jnp.float32)] * 5
    + [jax.ShapeDtypeStruct((_R, _C), jnp.int32)],
)


def _compact_body(x1h, y1h, x2h, y2h, msh, pmh, gout,
                  x1v, y1v, x2v, y2v, msv, pmv,
                  loc, idx2, cbuf, callv, counts_sh, sem):
    cid = lax.axis_index("c")
    t = lax.axis_index("s")

    @pl.when(cid == 0)
    def _work():
        off = t * _CH
        pltpu.sync_copy(x1h.at[pl.ds(off, _CH)], x1v)
        pltpu.sync_copy(y1h.at[pl.ds(off, _CH)], y1v)
        pltpu.sync_copy(x2h.at[pl.ds(off, _CH)], x2v)
        pltpu.sync_copy(y2h.at[pl.ds(off, _CH)], y2v)
        pltpu.sync_copy(msh.at[pl.ds(off, _CH)], msv)
        pltpu.sync_copy(pmh.at[pl.ds(off, _CH)], pmv)

        def cbody(i, acc):
            return acc + jnp.sum(pmv[pl.ds(i * 16, 16)])

        cnt = lax.fori_loop(0, _CH // 16, cbody, jnp.int32(0))

        cbuf[...] = jnp.broadcast_to(cnt, (16,))
        pltpu.sync_copy(cbuf, counts_sh.at[pl.ds(t * 16, 16)])
        plsc.subcore_barrier()
        pltpu.sync_copy(counts_sh, callv)

        def bbody(j, acc):
            v = callv[pl.ds(j * 16, 16)]
            return acc + jnp.where(j < t, v[0], 0)

        base = lax.fori_loop(0, 16, bbody, jnp.int32(0))

        lane = lax.iota(jnp.int32, 16)
        kcol = [jnp.full((16,), k, jnp.int32) for k in range(5)]

        def sbody(i, run):
            mi = pmv[pl.ds(i * 16, 16)]
            mb = mi > 0
            pc = plsc.cumsum(mi)
            gidx = jnp.where(mb, run + pc - 1, _TRASH + t)
            j = i * 16 + lane
            plsc.store_scatter(idx2, [j >> 7, j & 127], gidx)
            vals = (x1v, y1v, x2v, y2v, msv)
            for k in range(5):
                plsc.store_scatter(loc, [j, kcol[k]], vals[k][pl.ds(i * 16, 16)])
            return run + jnp.max(pc)

        lax.fori_loop(0, _CH // 16, sbody, base)

        for c in range(_CH // 128):
            pltpu.async_copy(loc.at[pl.ds(c * 128, 128)],
                             gout.at[idx2.at[c]], sem).wait()


@functools.cache
def _get_compact_call():
    return functools.partial(
        pl.kernel,
        mesh=plsc.VectorSubcoreMesh(core_axis_name="c", subcore_axis_name="s"),
        out_type=jax.ShapeDtypeStruct((_GROWS, 16), jnp.float32),
        compiler_params=pltpu.CompilerParams(
            needs_layout_passes=False, use_tc_tiling_on_sc=False),
            scratch_types=[
            pltpu.VMEM((_CH,), jnp.float32),
            pltpu.VMEM((_CH,), jnp.float32),
            pltpu.VMEM((_CH,), jnp.float32),
            pltpu.VMEM((_CH,), jnp.float32),
            pltpu.VMEM((_CH,), jnp.float32),
            pltpu.VMEM((_CH,), jnp.int32),
            pltpu.VMEM((_CH, 16), jnp.float32),
            pltpu.VMEM((_CH // 128, 128), jnp.int32),
            pltpu.VMEM((16,), jnp.int32),
            pltpu.VMEM((256,), jnp.int32),
            pltpu.VMEM_SHARED((256,), jnp.int32),
            pltpu.SemaphoreType.DMA,
        ],
    )(_compact_body)


def _nms_body(x1r, y1r, x2r, y2r, sr,
              ox1, oy1, ox2, oy2, osc):
    row = lax.broadcasted_iota(jnp.int32, (16, _C), 0)
    col = lax.broadcasted_iota(jnp.int32, (16, _C), 1)
    idx = row * _C + col

    cur0 = jnp.where(idx < _PRE, sr[...], _NEG)

    zero8 = jnp.zeros((8, _C), jnp.float32)
    ox1[...] = zero8
    oy1[...] = zero8
    ox2[...] = zero8
    oy2[...] = zero8
    osc[...] = zero8

    orow = lax.broadcasted_iota(jnp.int32, (8, _C), 0)
    ocol = lax.broadcasted_iota(jnp.int32, (8, _C), 1)
    oidx = orow * _C + ocol

    X1 = x1r[...]
    Y1 = y1r[...]
    X2 = x2r[...]
    Y2 = y2r[...]
    AR = (X2 - X1) * (Y2 - Y1)

    half = lambda a: (a[0:8, :], a[8:16, :])
    idx_t, idx_b = half(idx)
    x1_t, x1_b = half(X1)
    y1_t, y1_b = half(Y1)
    x2_t, x2_b = half(X2)
    y2_t, y2_b = half(Y2)

    def step(tt, cur):
        kt, kb = half(cur)
        c = (kb > kt) | ((kb == kt) & (idx_b < idx_t))
        k = jnp.where(c, kb, kt)
        i = jnp.where(c, idx_b, idx_t)
        bx1 = jnp.where(c, x1_b, x1_t)
        by1 = jnp.where(c, y1_b, y1_t)
        bx2 = jnp.where(c, x2_b, x2_t)
        by2 = jnp.where(c, y2_b, y2_t)

        for axis, shifts in ((0, (4, 2, 1)), (1, (64, 32, 16, 8, 4, 2, 1))):
            for sh in shifts:
                k2 = pltpu.roll(k, sh, axis)
                i2 = pltpu.roll(i, sh, axis)
                c = (k2 > k) | ((k2 == k) & (i2 < i))
                bx1 = jnp.where(c, pltpu.roll(bx1, sh, axis), bx1)
                by1 = jnp.where(c, pltpu.roll(by1, sh, axis), by1)
                bx2 = jnp.where(c, pltpu.roll(bx2, sh, axis), bx2)
                by2 = jnp.where(c, pltpu.roll(by2, sh, axis), by2)
                k = jnp.where(c, k2, k)
                i = jnp.where(c, i2, i)

        w = lambda a: jnp.concatenate([a, a], axis=0)
        wx1 = w(bx1)
        wy1 = w(by1)
        wx2 = w(bx2)
        wy2 = w(by2)

        ix1 = jnp.maximum(wx1, X1)
        iy1 = jnp.maximum(wy1, Y1)
        ix2 = jnp.minimum(wx2, X2)
        iy2 = jnp.minimum(wy2, Y2)
        inter = jnp.maximum(ix2 - ix1, 0.0) * jnp.maximum(iy2 - iy1, 0.0)
        area1 = (wx2 - wx1) * (wy2 - wy1)
        union = area1 + AR - inter
        iou = inter / jnp.maximum(union, 1e-6)
        supp = (iou >= _TH) | (idx == w(i))
        new_cur = jnp.where(supp, _NEG, cur)

        ok = k > (_NEG / 2.0)
        hit = oidx == tt
        ox1[...] = jnp.where(hit, jnp.where(ok, bx1, 0.0), ox1[...])
        oy1[...] = jnp.where(hit, jnp.where(ok, by1, 0.0), oy1[...])
        ox2[...] = jnp.where(hit, jnp.where(ok, bx2, 0.0), ox2[...])
        oy2[...] = jnp.where(hit, jnp.where(ok, by2, 0.0), oy2[...])
        osc[...] = jnp.where(hit, jnp.where(ok, k, 0.0), osc[...])
        return new_cur

    lax.fori_loop(0, _POST, step, cur0)


_nms_call = pl.pallas_call(
    _nms_body,
    out_shape=[jax.ShapeDtypeStruct((8, _C), jnp.float32)] * 5,
)


def kernel(anchors, pred_deltas, scores):
    pad = _PAD - _N

    def colf(x, i):
        return jnp.pad(x[:, i], (0, pad)).reshape(_R, _C)

    a = [colf(anchors, i) for i in range(4)]
    d = [colf(pred_deltas, i) for i in range(4)]
    s = jnp.pad(scores, (0, pad)).reshape(_R, _C)

    x1, y1, x2, y2, ms, pm = _prep_call(*a, *d, s)

    g = _get_compact_call()(x1.reshape(-1), y1.reshape(-1), x2.reshape(-1),
                            y2.reshape(-1), ms.reshape(-1), pm.reshape(-1))

    gr = g[:_SLOTS]
    planes = [gr[:, k].reshape(16, _C) for k in range(5)]

    outs = _nms_call(*planes)
    cols = [o.reshape(-1)[:_POST] for o in outs]
    return jnp.stack(cols, axis=1)


# speedup vs baseline: 1.1161x; 1.1161x over previous
import functools

import jax
import jax.numpy as jnp
from jax import lax
from jax.experimental import pallas as pl
from jax.experimental.pallas import tpu as pltpu
from jax.experimental.pallas import tpu_sc as plsc

_N = 20000
_PAD = 20480
_R = 160
_C = 128
_PRE = 2000
_POST = 1000
_TH = 0.7
_NEG = -1e30
_IMG_H = 1024.0
_IMG_W = 1024.0
_INT_MIN = -2147483648
_INT_MAX = 2147483647

_CH = _PAD // 16
_SLOTS = 2048
_TRASH = _SLOTS
_GROWS = _SLOTS + 16


def _prep_body(a0, a1, a2, a3, d0, d1, d2, d3, s,
               x1o, y1o, x2o, y2o, mso, poolo):
    A0, A1, A2, A3 = a0[...], a1[...], a2[...], a3[...]
    D0, D1, D2, D3 = d0[...], d1[...], d2[...], d3[...]
    S = s[...]

    ctr_x = (A0 + A2) / 2.0
    ctr_y = (A1 + A3) / 2.0
    w = A2 - A0
    h = A3 - A1
    x1 = jnp.minimum(jnp.maximum(ctr_x - D0 * w, 0.0), _IMG_W)
    y1 = jnp.minimum(jnp.maximum(ctr_y - D1 * h, 0.0), _IMG_H)
    x2 = jnp.minimum(jnp.maximum(ctr_x + D2 * w, 0.0), _IMG_W)
    y2 = jnp.minimum(jnp.maximum(ctr_y + D3 * h, 0.0), _IMG_H)

    row = lax.broadcasted_iota(jnp.int32, (_R, _C), 0)
    col = lax.broadcasted_iota(jnp.int32, (_R, _C), 1)
    idx = row * _C + col
    inb = idx < _N

    valid = (x2 - x1 > 0.0) & (y2 - y1 > 0.0) & inb
    ms = jnp.where(valid, S, _NEG)

    b = lax.bitcast_convert_type(ms, jnp.int32)
    key = jnp.where(b >= 0, b, _INT_MIN - b)

    def bs1(_, lohi):
        lo, hi = lohi
        mid = (lo >> 1) + (hi >> 1) + (lo & hi & 1)
        cnt = jnp.sum((inb & (key > mid)).astype(jnp.int32))
        go_hi = cnt < _PRE
        live = lo < hi
        new_lo = jnp.where(live & ~go_hi, mid + 1, lo)
        new_hi = jnp.where(live & go_hi, mid, hi)
        return new_lo, new_hi

    tau, _unused_hi = lax.fori_loop(
        0, 32, bs1, (jnp.int32(_INT_MIN), jnp.int32(_INT_MAX)))

    n_gt = jnp.sum((inb & (key > tau)).astype(jnp.int32))
    quota = _PRE - n_gt
    tie = inb & (key == tau)

    def bs2(_, lohi):
        lo, hi = lohi
        mid = (lo + hi) // 2
        cnt = jnp.sum((tie & (idx < mid)).astype(jnp.int32))
        go_hi = cnt >= quota
        live = lo < hi
        new_lo = jnp.where(live & ~go_hi, mid + 1, lo)
        new_hi = jnp.where(live & go_hi, mid, hi)
        return new_lo, new_hi

    mcut, _unused_hi2 = lax.fori_loop(0, 15, bs2, (jnp.int32(0), jnp.int32(_PAD)))

    pool = inb & ((key > tau) | (tie & (idx < mcut)))

    x1o[...] = x1
    y1o[...] = y1
    x2o[...] = x2
    y2o[...] = y2
    mso[...] = ms
    poolo[...] = pool.astype(jnp.int32)


_prep_call = pl.pallas_call(
    _prep_body,
    out_shape=[jax.ShapeDtypeStruct((_R, _C), jnp.float32)] * 5
    + [jax.ShapeDtypeStruct((_R, _C), jnp.int32)],
)


def _compact_body(x1h, y1h, x2h, y2h, msh, pmh, gout,
                  x1v, y1v, x2v, y2v, msv, pmv,
                  loc, idx2, cbuf, callv, counts_sh, sem):
    cid = lax.axis_index("c")
    t = lax.axis_index("s")

    @pl.when(cid == 0)
    def _work():
        off = t * _CH
        pltpu.sync_copy(x1h.at[pl.ds(off, _CH)], x1v)
        pltpu.sync_copy(y1h.at[pl.ds(off, _CH)], y1v)
        pltpu.sync_copy(x2h.at[pl.ds(off, _CH)], x2v)
        pltpu.sync_copy(y2h.at[pl.ds(off, _CH)], y2v)
        pltpu.sync_copy(msh.at[pl.ds(off, _CH)], msv)
        pltpu.sync_copy(pmh.at[pl.ds(off, _CH)], pmv)

        def cbody(i, acc):
            return acc + jnp.sum(pmv[pl.ds(i * 16, 16)])

        cnt = lax.fori_loop(0, _CH // 16, cbody, jnp.int32(0))

        cbuf[...] = jnp.broadcast_to(cnt, (16,))
        pltpu.sync_copy(cbuf, counts_sh.at[pl.ds(t * 16, 16)])
        plsc.subcore_barrier()
        pltpu.sync_copy(counts_sh, callv)

        def bbody(j, acc):
            v = callv[pl.ds(j * 16, 16)]
            return acc + jnp.where(j < t, v[0], 0)

        base = lax.fori_loop(0, 16, bbody, jnp.int32(0))

        lane = lax.iota(jnp.int32, 16)
        kcol = [jnp.full((16,), k, jnp.int32) for k in range(5)]

        def sbody(i, run):
            mi = pmv[pl.ds(i * 16, 16)]
            mb = mi > 0
            pc = plsc.cumsum(mi)
            gidx = jnp.where(mb, run + pc - 1, _TRASH + t)
            j = i * 16 + lane
            plsc.store_scatter(idx2, [j >> 7, j & 127], gidx)
            vals = (x1v, y1v, x2v, y2v, msv)
            for k in range(5):
                plsc.store_scatter(loc, [j, kcol[k]], vals[k][pl.ds(i * 16, 16)])
            return run + jnp.max(pc)

        lax.fori_loop(0, _CH // 16, sbody, base)

        for c in range(_CH // 128):
            pltpu.async_copy(loc.at[pl.ds(c * 128, 128)],
                             gout.at[idx2.at[c]], sem).wait()


@functools.cache
def _get_compact_call():
    return functools.partial(
        pl.kernel,
        mesh=plsc.VectorSubcoreMesh(core_axis_name="c", subcore_axis_name="s"),
        out_type=jax.ShapeDtypeStruct((_GROWS, 16), jnp.float32),
        compiler_params=pltpu.CompilerParams(
            needs_layout_passes=False, use_tc_tiling_on_sc=False),
            scratch_types=[
            pltpu.VMEM((_CH,), jnp.float32),
            pltpu.VMEM((_CH,), jnp.float32),
            pltpu.VMEM((_CH,), jnp.float32),
            pltpu.VMEM((_CH,), jnp.float32),
            pltpu.VMEM((_CH,), jnp.float32),
            pltpu.VMEM((_CH,), jnp.int32),
            pltpu.VMEM((_CH, 16), jnp.float32),
            pltpu.VMEM((_CH // 128, 128), jnp.int32),
            pltpu.VMEM((16,), jnp.int32),
            pltpu.VMEM((256,), jnp.int32),
            pltpu.VMEM_SHARED((256,), jnp.int32),
            pltpu.SemaphoreType.DMA,
        ],
    )(_compact_body)


_NR = 16
_NL = _C
_OR = 8


def _nms_body(x1r, y1r, x2r, y2r, sr,
              ox1, oy1, ox2, oy2, osc,
              curr):
    row = lax.broadcasted_iota(jnp.int32, (_NR, _C), 0)
    col = lax.broadcasted_iota(jnp.int32, (_NR, _C), 1)
    idx = row * _C + col

    curr[...] = jnp.where(idx < _PRE, sr[...], _NEG)

    zero8 = jnp.zeros((_OR, _C), jnp.float32)
    ox1[...] = zero8
    oy1[...] = zero8
    ox2[...] = zero8
    oy2[...] = zero8
    osc[...] = zero8

    orow = lax.broadcasted_iota(jnp.int32, (_OR, _C), 0)
    ocol = lax.broadcasted_iota(jnp.int32, (_OR, _C), 1)
    oidx = orow * _C + ocol

    lane = lax.broadcasted_iota(jnp.int32, (1, _C), 1)

    X1 = x1r[...]
    Y1 = y1r[...]
    X2 = x2r[...]
    Y2 = y2r[...]
    AR = (X2 - X1) * (Y2 - Y1)

    def step(tt, carry):
        cur = curr[...]
        m = jnp.max(cur)
        j = jnp.min(jnp.where(cur == m, idx, _SLOTS))

        r = j >> 7
        c = j & 127
        lm = lane == c
        bx1 = jnp.max(jnp.where(lm, x1r[pl.ds(r, 1), :], _NEG))
        by1 = jnp.max(jnp.where(lm, y1r[pl.ds(r, 1), :], _NEG))
        bx2 = jnp.max(jnp.where(lm, x2r[pl.ds(r, 1), :], _NEG))
        by2 = jnp.max(jnp.where(lm, y2r[pl.ds(r, 1), :], _NEG))

        ix1 = jnp.maximum(bx1, X1)
        iy1 = jnp.maximum(by1, Y1)
        ix2 = jnp.minimum(bx2, X2)
        iy2 = jnp.minimum(by2, Y2)
        inter = jnp.maximum(ix2 - ix1, 0.0) * jnp.maximum(iy2 - iy1, 0.0)
        area1 = (bx2 - bx1) * (by2 - by1)
        union = area1 + AR - inter
        iou = inter / jnp.maximum(union, 1e-6)
        supp = (iou >= _TH) | (idx == j)
        curr[...] = jnp.where(supp, _NEG, cur)

        ok = m > (_NEG / 2.0)
        hit = oidx == tt
        ox1[...] = jnp.where(hit, jnp.where(ok, bx1, 0.0), ox1[...])
        oy1[...] = jnp.where(hit, jnp.where(ok, by1, 0.0), oy1[...])
        ox2[...] = jnp.where(hit, jnp.where(ok, bx2, 0.0), ox2[...])
        oy2[...] = jnp.where(hit, jnp.where(ok, by2, 0.0), oy2[...])
        osc[...] = jnp.where(hit, jnp.where(ok, m, 0.0), osc[...])
        return carry

    lax.fori_loop(0, _POST, step, 0)


_nms_call = pl.pallas_call(
    _nms_body,
    out_shape=[jax.ShapeDtypeStruct((_OR, _C), jnp.float32)] * 5,
    scratch_shapes=[pltpu.VMEM((_NR, _C), jnp.float32)],
)


def kernel(anchors, pred_deltas, scores):
    pad = _PAD - _N

    def colf(x, i):
        return jnp.pad(x[:, i], (0, pad)).reshape(_R, _C)

    a = [colf(anchors, i) for i in range(4)]
    d = [colf(pred_deltas, i) for i in range(4)]
    s = jnp.pad(scores, (0, pad)).reshape(_R, _C)

    x1, y1, x2, y2, ms, pm = _prep_call(*a, *d, s)

    g = _get_compact_call()(x1.reshape(-1), y1.reshape(-1), x2.reshape(-1),
                            y2.reshape(-1), ms.reshape(-1), pm.reshape(-1))

    gr = g[:_SLOTS]
    planes = [gr[:, k].reshape(_NR, _NL) for k in range(5)]

    outs = _nms_call(*planes)
    cols = [o.reshape(-1)[:_POST] for o in outs]
    return jnp.stack(cols, axis=1)


# speedup vs baseline: 1.4093x; 1.2628x over previous
import functools

import jax
import jax.numpy as jnp
from jax import lax
from jax.experimental import pallas as pl
from jax.experimental.pallas import tpu as pltpu
from jax.experimental.pallas import tpu_sc as plsc

_N = 20000
_PAD = 20480
_R = 160
_C = 128
_PRE = 2000
_POST = 1000
_TH = 0.7
_NEG = -1e30
_IMG_H = 1024.0
_IMG_W = 1024.0
_INT_MIN = -2147483648
_INT_MAX = 2147483647

_CH = _PAD // 16
_SLOTS = 2048
_TRASH = _SLOTS
_GROWS = _SLOTS + 16


def _prep_body(a0, a1, a2, a3, d0, d1, d2, d3, s,
               x1o, y1o, x2o, y2o, mso, poolo):
    A0, A1, A2, A3 = a0[...], a1[...], a2[...], a3[...]
    D0, D1, D2, D3 = d0[...], d1[...], d2[...], d3[...]
    S = s[...]

    ctr_x = (A0 + A2) / 2.0
    ctr_y = (A1 + A3) / 2.0
    w = A2 - A0
    h = A3 - A1
    x1 = jnp.minimum(jnp.maximum(ctr_x - D0 * w, 0.0), _IMG_W)
    y1 = jnp.minimum(jnp.maximum(ctr_y - D1 * h, 0.0), _IMG_H)
    x2 = jnp.minimum(jnp.maximum(ctr_x + D2 * w, 0.0), _IMG_W)
    y2 = jnp.minimum(jnp.maximum(ctr_y + D3 * h, 0.0), _IMG_H)

    row = lax.broadcasted_iota(jnp.int32, (_R, _C), 0)
    col = lax.broadcasted_iota(jnp.int32, (_R, _C), 1)
    idx = row * _C + col
    inb = idx < _N

    valid = (x2 - x1 > 0.0) & (y2 - y1 > 0.0) & inb
    ms = jnp.where(valid, S, _NEG)

    b = lax.bitcast_convert_type(ms, jnp.int32)
    key = jnp.where(b >= 0, b, _INT_MIN - b)

    def bs1(_, lohi):
        lo, hi = lohi
        mid = (lo >> 1) + (hi >> 1) + (lo & hi & 1)
        cnt = jnp.sum((inb & (key > mid)).astype(jnp.int32))
        go_hi = cnt < _PRE
        live = lo < hi
        new_lo = jnp.where(live & ~go_hi, mid + 1, lo)
        new_hi = jnp.where(live & go_hi, mid, hi)
        return new_lo, new_hi

    tau, _unused_hi = lax.fori_loop(
        0, 32, bs1, (jnp.int32(_INT_MIN), jnp.int32(_INT_MAX)))

    n_gt = jnp.sum((inb & (key > tau)).astype(jnp.int32))
    quota = _PRE - n_gt
    tie = inb & (key == tau)

    def bs2(_, lohi):
        lo, hi = lohi
        mid = (lo + hi) // 2
        cnt = jnp.sum((tie & (idx < mid)).astype(jnp.int32))
        go_hi = cnt >= quota
        live = lo < hi
        new_lo = jnp.where(live & ~go_hi, mid + 1, lo)
        new_hi = jnp.where(live & go_hi, mid, hi)
        return new_lo, new_hi

    mcut, _unused_hi2 = lax.fori_loop(0, 15, bs2, (jnp.int32(0), jnp.int32(_PAD)))

    pool = inb & ((key > tau) | (tie & (idx < mcut)))

    x1o[...] = x1
    y1o[...] = y1
    x2o[...] = x2
    y2o[...] = y2
    mso[...] = ms
    poolo[...] = pool.astype(jnp.int32)


_prep_call = pl.pallas_call(
    _prep_body,
    out_shape=[jax.ShapeDtypeStruct((_R, _C), jnp.float32)] * 5
    + [jax.ShapeDtypeStruct((_R, _C), jnp.int32)],
)


def _compact_body(x1h, y1h, x2h, y2h, msh, pmh, gout,
                  x1v, y1v, x2v, y2v, msv, pmv,
                  loc, idx2, cbuf, callv, counts_sh, sem):
    cid = lax.axis_index("c")
    t = lax.axis_index("s")

    @pl.when(cid == 0)
    def _work():
        off = t * _CH
        pltpu.sync_copy(x1h.at[pl.ds(off, _CH)], x1v)
        pltpu.sync_copy(y1h.at[pl.ds(off, _CH)], y1v)
        pltpu.sync_copy(x2h.at[pl.ds(off, _CH)], x2v)
        pltpu.sync_copy(y2h.at[pl.ds(off, _CH)], y2v)
        pltpu.sync_copy(msh.at[pl.ds(off, _CH)], msv)
        pltpu.sync_copy(pmh.at[pl.ds(off, _CH)], pmv)

        def cbody(i, acc):
            return acc + jnp.sum(pmv[pl.ds(i * 16, 16)])

        cnt = lax.fori_loop(0, _CH // 16, cbody, jnp.int32(0))

        cbuf[...] = jnp.broadcast_to(cnt, (16,))
        pltpu.sync_copy(cbuf, counts_sh.at[pl.ds(t * 16, 16)])
        plsc.subcore_barrier()
        pltpu.sync_copy(counts_sh, callv)

        def bbody(j, acc):
            v = callv[pl.ds(j * 16, 16)]
            return acc + jnp.where(j < t, v[0], 0)

        base = lax.fori_loop(0, 16, bbody, jnp.int32(0))

        lane = lax.iota(jnp.int32, 16)
        kcol = [jnp.full((16,), k, jnp.int32) for k in range(5)]

        def sbody(i, run):
            mi = pmv[pl.ds(i * 16, 16)]
            mb = mi > 0
            pc = plsc.cumsum(mi)
            gidx = jnp.where(mb, run + pc - 1, _TRASH + t)
            j = i * 16 + lane
            plsc.store_scatter(idx2, [j >> 7, j & 127], gidx)
            vals = (x1v, y1v, x2v, y2v, msv)
            for k in range(5):
                plsc.store_scatter(loc, [j, kcol[k]], vals[k][pl.ds(i * 16, 16)])
            return run + jnp.max(pc)

        lax.fori_loop(0, _CH // 16, sbody, base)

        for c in range(_CH // 128):
            pltpu.async_copy(loc.at[pl.ds(c * 128, 128)],
                             gout.at[idx2.at[c]], sem).wait()


@functools.cache
def _get_compact_call():
    return functools.partial(
        pl.kernel,
        mesh=plsc.VectorSubcoreMesh(core_axis_name="c", subcore_axis_name="s"),
        out_type=jax.ShapeDtypeStruct((_GROWS, 16), jnp.float32),
        compiler_params=pltpu.CompilerParams(
            needs_layout_passes=False, use_tc_tiling_on_sc=False),
            scratch_types=[
            pltpu.VMEM((_CH,), jnp.float32),
            pltpu.VMEM((_CH,), jnp.float32),
            pltpu.VMEM((_CH,), jnp.float32),
            pltpu.VMEM((_CH,), jnp.float32),
            pltpu.VMEM((_CH,), jnp.float32),
            pltpu.VMEM((_CH,), jnp.int32),
            pltpu.VMEM((_CH, 16), jnp.float32),
            pltpu.VMEM((_CH // 128, 128), jnp.int32),
            pltpu.VMEM((16,), jnp.int32),
            pltpu.VMEM((256,), jnp.int32),
            pltpu.VMEM_SHARED((256,), jnp.int32),
            pltpu.SemaphoreType.DMA,
        ],
    )(_compact_body)


_NR = 16
_NL = _C
_OR = 8


def _nms_body(x1r, y1r, x2r, y2r, sr,
              x1b, y1b, x2b, y2b,
              ox1, oy1, ox2, oy2, osc,
              curr):
    row = lax.broadcasted_iota(jnp.int32, (_NR, _C), 0)
    col = lax.broadcasted_iota(jnp.int32, (_NR, _C), 1)
    idx = row * _C + col

    curr[...] = jnp.where(idx < _PRE, sr[...], _NEG)

    zero8 = jnp.zeros((_OR, _C), jnp.float32)
    ox1[...] = zero8
    oy1[...] = zero8
    ox2[...] = zero8
    oy2[...] = zero8
    osc[...] = zero8

    orow = lax.broadcasted_iota(jnp.int32, (_OR, _C), 0)
    ocol = lax.broadcasted_iota(jnp.int32, (_OR, _C), 1)
    oidx = orow * _C + ocol

    X1 = x1r[...]
    Y1 = y1r[...]
    X2 = x2r[...]
    Y2 = y2r[...]
    AR = (X2 - X1) * (Y2 - Y1)

    def step(tt, carry):
        cur = curr[...]
        m = jnp.max(cur)
        j = jnp.min(jnp.where(cur == m, idx, _SLOTS))

        bx1 = x1b[pl.ds(j, 1), :]
        by1 = y1b[pl.ds(j, 1), :]
        bx2 = x2b[pl.ds(j, 1), :]
        by2 = y2b[pl.ds(j, 1), :]

        ix1 = jnp.maximum(bx1, X1)
        iy1 = jnp.maximum(by1, Y1)
        ix2 = jnp.minimum(bx2, X2)
        iy2 = jnp.minimum(by2, Y2)
        inter = jnp.maximum(ix2 - ix1, 0.0) * jnp.maximum(iy2 - iy1, 0.0)
        area1 = (bx2 - bx1) * (by2 - by1)
        union = area1 + AR - inter
        iou = inter / jnp.maximum(union, 1e-6)
        supp = (iou >= _TH) | (idx == j)
        curr[...] = jnp.where(supp, _NEG, cur)

        ok = m > (_NEG / 2.0)
        hit = oidx == tt
        ox1[...] = jnp.where(hit, jnp.where(ok, bx1, 0.0), ox1[...])
        oy1[...] = jnp.where(hit, jnp.where(ok, by1, 0.0), oy1[...])
        ox2[...] = jnp.where(hit, jnp.where(ok, bx2, 0.0), ox2[...])
        oy2[...] = jnp.where(hit, jnp.where(ok, by2, 0.0), oy2[...])
        osc[...] = jnp.where(hit, jnp.where(ok, m, 0.0), osc[...])
        return carry

    lax.fori_loop(0, _POST, step, 0)


_nms_call = pl.pallas_call(
    _nms_body,
    out_shape=[jax.ShapeDtypeStruct((_OR, _C), jnp.float32)] * 5,
    scratch_shapes=[pltpu.VMEM((_NR, _C), jnp.float32)],
)


def kernel(anchors, pred_deltas, scores):
    pad = _PAD - _N

    def colf(x, i):
        return jnp.pad(x[:, i], (0, pad)).reshape(_R, _C)

    a = [colf(anchors, i) for i in range(4)]
    d = [colf(pred_deltas, i) for i in range(4)]
    s = jnp.pad(scores, (0, pad)).reshape(_R, _C)

    x1, y1, x2, y2, ms, pm = _prep_call(*a, *d, s)

    g = _get_compact_call()(x1.reshape(-1), y1.reshape(-1), x2.reshape(-1),
                            y2.reshape(-1), ms.reshape(-1), pm.reshape(-1))

    gr = g[:_SLOTS]
    planes = [gr[:, k].reshape(_NR, _NL) for k in range(5)]
    bplanes = [jnp.broadcast_to(gr[:, k][:, None], (_SLOTS, _C))
               for k in range(4)]

    outs = _nms_call(*planes, *bplanes)
    cols = [o.reshape(-1)[:_POST] for o in outs]
    return jnp.stack(cols, axis=1)
